# baseline (speedup 1.0000x reference)
import functools

import jax
import jax.numpy as jnp
from jax.experimental import pallas as pl
from jax.experimental.pallas import tpu as pltpu


def _body(targ_ref, x_ref, ll_ref, loss_ref, lams_ref, cnt_ref,
          ll16_ref, lsum_ref, acc_ref, *, b_total):
    i = pl.program_id(0)
    nb = pl.num_programs(0)
    x = x_ref[...]
    t = targ_ref[0, 0, :]
    bb, c = x.shape

    @pl.when(i == 0)
    def _prep():
        ll16_ref[...] = ll_ref[...].astype(jnp.bfloat16)

    m = jnp.max(x, axis=1, keepdims=True)
    ex = jnp.exp(x - m)
    se = jnp.sum(ex, axis=1, keepdims=True)
    lse = m + jnp.log(se)

    cidx = jax.lax.broadcasted_iota(jnp.int32, (bb, c), 1)
    top1 = jnp.min(jnp.where(x == m, cidx, c), axis=1)
    corr = (t == top1).astype(jnp.float32)[:, None]
    oh = cidx == t[:, None]
    oh16 = oh.astype(jnp.bfloat16)

    e16 = (ex * (corr / se)).astype(jnp.bfloat16)
    lam_blk = jax.lax.dot_general(
        oh16, e16, (((0,), (0,)), ((), ())),
        preferred_element_type=jnp.float32)
    a_blk = jax.lax.dot_general(
        x.astype(jnp.bfloat16), ll16_ref[...], (((1,), (1,)), ((), ())),
        preferred_element_type=jnp.float32)

    lsum_blk = jnp.sum(jnp.where(oh, lse, 0.0), axis=0)[None, :]
    acc_blk = jnp.sum(lse) - jnp.sum(jnp.where(oh, x + a_blk, 0.0))

    @pl.when(i == 0)
    def _init():
        lams_ref[...] = lam_blk
        lsum_ref[...] = lsum_blk
        acc_ref[...] = jnp.full((1, 1), acc_blk, jnp.float32)

    @pl.when(i > 0)
    def _acc():
        lams_ref[...] += lam_blk
        lsum_ref[...] += lsum_blk
        acc_ref[...] += jnp.full((1, 1), acc_blk, jnp.float32)

    @pl.when(i == nb - 1)
    def _fin():
        rs = jnp.sum(ll_ref[...], axis=1)[None, :]
        soft_extra = jnp.sum(rs * lsum_ref[...])
        cnt_ref[...] = jnp.sum(lams_ref[...], axis=1)[None, :]
        val = 0.5 * (acc_ref[0, 0] + soft_extra) / b_total
        loss_ref[...] = jnp.full((1, 1), val, jnp.float32)


def kernel(output, target, loss_lams):
    bn, cn = output.shape
    bb = 1024
    nb = bn // bb
    targ3 = target.reshape(nb, 1, bb)

    loss, lams, cnt = pl.pallas_call(
        functools.partial(_body, b_total=bn),
        grid=(nb,),
        in_specs=[
            pl.BlockSpec((1, 1, bb), lambda i: (i, 0, 0)),
            pl.BlockSpec((bb, cn), lambda i: (i, 0)),
            pl.BlockSpec((cn, cn), lambda i: (0, 0)),
        ],
        out_specs=[
            pl.BlockSpec((1, 1), lambda i: (0, 0)),
            pl.BlockSpec((cn, cn), lambda i: (0, 0)),
            pl.BlockSpec((1, cn), lambda i: (0, 0)),
        ],
        out_shape=[
            jax.ShapeDtypeStruct((1, 1), jnp.float32),
            jax.ShapeDtypeStruct((cn, cn), jnp.float32),
            jax.ShapeDtypeStruct((1, cn), jnp.float32),
        ],
        scratch_shapes=[
            pltpu.VMEM((cn, cn), jnp.bfloat16),
            pltpu.VMEM((1, cn), jnp.float32),
            pltpu.VMEM((1, 1), jnp.float32),
        ],
    )(targ3, output, loss_lams)

    return loss[0, 0], lams, cnt[0]


# speedup vs baseline: 1.1913x; 1.1913x over previous
import functools

import jax
import jax.numpy as jnp
from jax.experimental import pallas as pl
from jax.experimental.pallas import tpu as pltpu


def _body(targ_ref, x_ref, ll_ref, loss_ref, lams_ref, cnt_ref,
          ll16_ref, lsum_ref, acc_ref, *, b_total):
    i = pl.program_id(0)
    nb = pl.num_programs(0)
    x = x_ref[...]
    t = targ_ref[0, 0, :]
    bb, c = x.shape

    @pl.when(i == 0)
    def _prep():
        ll16_ref[...] = ll_ref[...].astype(jnp.bfloat16)

    a_blk = jax.lax.dot_general(
        x.astype(jnp.bfloat16), ll16_ref[...], (((1,), (1,)), ((), ())),
        preferred_element_type=jnp.float32)

    m = jnp.max(x, axis=1, keepdims=True)
    ex = jnp.exp(x - m)
    se = jnp.sum(ex, axis=1, keepdims=True)
    lse = m + jnp.log(se)

    cidx = jax.lax.broadcasted_iota(jnp.int32, (bb, c), 1)
    top1 = jnp.min(jnp.where(x == m, cidx, c), axis=1)
    corr = (t == top1).astype(jnp.float32)[:, None]
    oh = cidx == t[:, None]
    oh16 = oh.astype(jnp.bfloat16)

    e16 = (ex * (corr / se)).astype(jnp.bfloat16)
    lam_blk = jax.lax.dot_general(
        oh16, e16, (((0,), (0,)), ((), ())),
        preferred_element_type=jnp.float32)

    lsum_blk = jnp.sum(jnp.where(oh, lse, 0.0), axis=0)[None, :]
    acc_blk = jnp.sum(lse) - jnp.sum(jnp.where(oh, x + a_blk, 0.0))

    @pl.when(i == 0)
    def _init():
        lams_ref[...] = lam_blk
        lsum_ref[...] = lsum_blk
        acc_ref[...] = jnp.full((1, 1), acc_blk, jnp.float32)

    @pl.when(i > 0)
    def _acc():
        lams_ref[...] += lam_blk
        lsum_ref[...] += lsum_blk
        acc_ref[...] += jnp.full((1, 1), acc_blk, jnp.float32)

    @pl.when(i == nb - 1)
    def _fin():
        rs = jnp.sum(ll_ref[...], axis=1)[None, :]
        soft_extra = jnp.sum(rs * lsum_ref[...])
        cnt_ref[...] = jnp.sum(lams_ref[...], axis=1)[None, :]
        val = 0.5 * (acc_ref[0, 0] + soft_extra) / b_total
        loss_ref[...] = jnp.full((1, 1), val, jnp.float32)


def kernel(output, target, loss_lams):
    bn, cn = output.shape
    bb = 1024
    nb = bn // bb
    targ3 = target.reshape(nb, 1, bb)

    loss, lams, cnt = pl.pallas_call(
        functools.partial(_body, b_total=bn),
        grid=(nb,),
        in_specs=[
            pl.BlockSpec((1, 1, bb), lambda i: (i, 0, 0)),
            pl.BlockSpec((bb, cn), lambda i: (i, 0)),
            pl.BlockSpec((cn, cn), lambda i: (0, 0)),
        ],
        out_specs=[
            pl.BlockSpec((1, 1), lambda i: (0, 0)),
            pl.BlockSpec((cn, cn), lambda i: (0, 0)),
            pl.BlockSpec((1, cn), lambda i: (0, 0)),
        ],
        out_shape=[
            jax.ShapeDtypeStruct((1, 1), jnp.float32),
            jax.ShapeDtypeStruct((cn, cn), jnp.float32),
            jax.ShapeDtypeStruct((1, cn), jnp.float32),
        ],
        scratch_shapes=[
            pltpu.VMEM((cn, cn), jnp.bfloat16),
            pltpu.VMEM((1, cn), jnp.float32),
            pltpu.VMEM((1, 1), jnp.float32),
        ],
    )(targ3, output, loss_lams)

    return loss[0, 0], lams, cnt[0]
